# baseline (speedup 1.0000x reference)
import jax
import jax.numpy as jnp
from jax.experimental import pallas as pl
from jax.experimental.pallas import tpu as pltpu

BM = 2048
BK = 1024
CHUNK = 2736


def _vq_block_kernel(h_ref, cb_ref, hsq_ref, csq_ref, out_ref, m_ref, j_ref):
    k = pl.program_id(1)
    nk = pl.num_programs(1)

    scores = jax.lax.dot_general(
        h_ref[...], cb_ref[...], (((1,), (1,)), ((), ())),
        preferred_element_type=jnp.float32,
    )
    dist = (hsq_ref[...] + csq_ref[...]) - 2.0 * scores

    base = k * BK
    cid = (base + jax.lax.broadcasted_iota(jnp.int32, (1, BK), 1)) // CHUNK

    @pl.when(k == 0)
    def _init():
        m_ref[...] = jnp.full(m_ref.shape, jnp.inf, jnp.float32)
        j_ref[...] = jnp.zeros(j_ref.shape, jnp.int32)

    for c in range(3):
        dc = jnp.where(cid == c, dist, jnp.inf)
        bm = jnp.min(dc, axis=1)
        ba = jnp.argmin(dc, axis=1).astype(jnp.int32) + base
        prev = m_ref[c, :]
        upd = bm < prev
        m_ref[c, :] = jnp.where(upd, bm, prev)
        j_ref[c, :] = jnp.where(upd, ba, j_ref[c, :])

    @pl.when(k == nk - 1)
    def _emit():
        idx = j_ref[0, :]
        acc = m_ref[0, :].astype(jnp.bfloat16).astype(jnp.float32)
        for c in (1, 2):
            mc = m_ref[c, :]
            take = mc < acc
            idx = jnp.where(take, j_ref[c, :], idx)
            acc = jnp.where(take, mc.astype(jnp.bfloat16).astype(jnp.float32),
                            acc)
        out_ref[...] = idx[:, None]


def kernel(h, codebook):
    B, T, D = h.shape
    K = codebook.shape[0]
    M = B * T
    h_flat = h.reshape(M, D)

    h_sq = jnp.sum(h_flat ** 2, axis=-1, keepdims=True)
    c_sq = jnp.sum(codebook ** 2, axis=-1, keepdims=True).T
    h_bf = h_flat.astype(jnp.bfloat16)
    cb_bf = codebook.astype(jnp.bfloat16)

    num_m = M // BM
    num_k = K // BK

    out = pl.pallas_call(
        _vq_block_kernel,
        grid=(num_m, num_k),
        in_specs=[
            pl.BlockSpec((BM, D), lambda m, k: (m, 0)),
            pl.BlockSpec((BK, D), lambda m, k: (k, 0)),
            pl.BlockSpec((BM, 1), lambda m, k: (m, 0)),
            pl.BlockSpec((1, BK), lambda m, k: (0, k)),
        ],
        out_specs=pl.BlockSpec((BM, 1), lambda m, k: (m, 0)),
        out_shape=jax.ShapeDtypeStruct((M, 1), jnp.int32),
        scratch_shapes=[
            pltpu.VMEM((3, BM), jnp.float32),
            pltpu.VMEM((3, BM), jnp.int32),
        ],
        compiler_params=pltpu.CompilerParams(
            dimension_semantics=("parallel", "arbitrary"),
        ),
    )(h_bf, cb_bf, h_sq, c_sq)
    return out.reshape(B, T)


# speedup vs baseline: 1.7511x; 1.7511x over previous
import jax
import jax.numpy as jnp
from jax.experimental import pallas as pl
from jax.experimental.pallas import tpu as pltpu

BM = 2048
BK = 1024
CHUNK = 2736


_STRADDLE = {
    k: ((k * BK) // CHUNK, (k * BK + BK - 1) // CHUNK,
        CHUNK * ((k * BK) // CHUNK + 1) - k * BK)
    for k in range(8192 // BK)
    if (k * BK) // CHUNK != (k * BK + BK - 1) // CHUNK
}


def _vq_block_kernel(h_ref, cb_ref, hsq_ref, csq_ref, out_ref, m_ref, j_ref):
    k = pl.program_id(1)
    nk = pl.num_programs(1)

    scores = jax.lax.dot_general(
        h_ref[...], cb_ref[...], (((1,), (1,)), ((), ())),
        preferred_element_type=jnp.float32,
    )
    dist = (hsq_ref[...] + csq_ref[...]) - 2.0 * scores

    base = k * BK

    @pl.when(k == 0)
    def _init():
        m_ref[...] = jnp.full(m_ref.shape, jnp.inf, jnp.float32)
        j_ref[...] = jnp.zeros(j_ref.shape, jnp.int32)

    def _update(c, bm, ba, pred=None):
        prev = m_ref[c, :]
        upd = bm < prev
        if pred is not None:
            upd = jnp.logical_and(upd, pred)
        m_ref[c, :] = jnp.where(upd, bm, prev)
        j_ref[c, :] = jnp.where(upd, ba, j_ref[c, :])

    straddle_pred = jnp.zeros((), jnp.bool_)
    for kk, (c_lo, c_hi, off) in _STRADDLE.items():
        straddle_pred = jnp.logical_or(straddle_pred, k == kk)

        @pl.when(k == kk)
        def _straddle(c_lo=c_lo, c_hi=c_hi, off=off):
            dl = dist[:, :off]
            _update(c_lo, jnp.min(dl, axis=1),
                    jnp.argmin(dl, axis=1).astype(jnp.int32) + base)
            dr = dist[:, off:]
            _update(c_hi, jnp.min(dr, axis=1),
                    jnp.argmin(dr, axis=1).astype(jnp.int32) + (base + off))

    @pl.when(jnp.logical_not(straddle_pred))
    def _single():
        bm = jnp.min(dist, axis=1)
        ba = jnp.argmin(dist, axis=1).astype(jnp.int32) + base
        c_dyn = base // CHUNK
        for c in range(3):
            _update(c, bm, ba, pred=(c_dyn == c))

    @pl.when(k == nk - 1)
    def _emit():
        idx = j_ref[0, :]
        acc = m_ref[0, :].astype(jnp.bfloat16).astype(jnp.float32)
        for c in (1, 2):
            mc = m_ref[c, :]
            take = mc < acc
            idx = jnp.where(take, j_ref[c, :], idx)
            acc = jnp.where(take, mc.astype(jnp.bfloat16).astype(jnp.float32),
                            acc)
        out_ref[...] = idx[:, None]


def kernel(h, codebook):
    B, T, D = h.shape
    K = codebook.shape[0]
    M = B * T
    h_flat = h.reshape(M, D)

    h_sq = jnp.sum(h_flat ** 2, axis=-1, keepdims=True)
    c_sq = jnp.sum(codebook ** 2, axis=-1, keepdims=True).T
    h_bf = h_flat.astype(jnp.bfloat16)
    cb_bf = codebook.astype(jnp.bfloat16)

    num_m = M // BM
    num_k = K // BK

    out = pl.pallas_call(
        _vq_block_kernel,
        grid=(num_m, num_k),
        in_specs=[
            pl.BlockSpec((BM, D), lambda m, k: (m, 0)),
            pl.BlockSpec((BK, D), lambda m, k: (k, 0)),
            pl.BlockSpec((BM, 1), lambda m, k: (m, 0)),
            pl.BlockSpec((1, BK), lambda m, k: (0, k)),
        ],
        out_specs=pl.BlockSpec((BM, 1), lambda m, k: (m, 0)),
        out_shape=jax.ShapeDtypeStruct((M, 1), jnp.int32),
        scratch_shapes=[
            pltpu.VMEM((3, BM), jnp.float32),
            pltpu.VMEM((3, BM), jnp.int32),
        ],
        compiler_params=pltpu.CompilerParams(
            dimension_semantics=("parallel", "arbitrary"),
        ),
    )(h_bf, cb_bf, h_sq, c_sq)
    return out.reshape(B, T)


# speedup vs baseline: 2.5356x; 1.4480x over previous
import jax
import jax.numpy as jnp
from jax.experimental import pallas as pl
from jax.experimental.pallas import tpu as pltpu

BM = 1024
BK = 1024
CHUNK = 2736

_STRADDLE = {
    k: ((k * BK) // CHUNK, (k * BK + BK - 1) // CHUNK,
        CHUNK * ((k * BK) // CHUNK + 1) - k * BK)
    for k in range(8192 // BK)
    if (k * BK) // CHUNK != (k * BK + BK - 1) // CHUNK
}


def _vq_block_kernel(h_ref, cb_ref, hsq_ref, csq_ref, out_ref, m_ref, j_ref):
    k = pl.program_id(1)
    nk = pl.num_programs(1)

    scores = jax.lax.dot_general(
        h_ref[...], cb_ref[...], (((1,), (1,)), ((), ())),
        preferred_element_type=jnp.float32,
    )
    dist = (hsq_ref[...] + csq_ref[...]) - 2.0 * scores

    base = k * BK

    @pl.when(k == 0)
    def _init():
        m_ref[...] = jnp.full(m_ref.shape, jnp.inf, jnp.float32)
        j_ref[...] = jnp.zeros(j_ref.shape, jnp.int32)

    def _update(c, bm, ba, pred=None):
        prev = m_ref[:, c:c + 1]
        upd = bm < prev
        if pred is not None:
            upd = jnp.logical_and(upd, pred)
        m_ref[:, c:c + 1] = jnp.where(upd, bm, prev)
        j_ref[:, c:c + 1] = jnp.where(upd, ba, j_ref[:, c:c + 1])

    def _minarg(d, off):
        bm = jnp.min(d, axis=1, keepdims=True)
        ba = jnp.argmin(d, axis=1, keepdims=True).astype(jnp.int32) + off
        return bm, ba

    straddle_pred = jnp.zeros((), jnp.bool_)
    for kk, (c_lo, c_hi, off) in _STRADDLE.items():
        straddle_pred = jnp.logical_or(straddle_pred, k == kk)

        @pl.when(k == kk)
        def _straddle(c_lo=c_lo, c_hi=c_hi, off=off):
            bm, ba = _minarg(dist[:, :off], base)
            _update(c_lo, bm, ba)
            bm, ba = _minarg(dist[:, off:], base + off)
            _update(c_hi, bm, ba)

    @pl.when(jnp.logical_not(straddle_pred))
    def _single():
        bm, ba = _minarg(dist, base)
        c_dyn = base // CHUNK
        for c in range(3):
            _update(c, bm, ba, pred=(c_dyn == c))

    @pl.when(k == nk - 1)
    def _emit():
        idx = j_ref[:, 0:1]
        acc = m_ref[:, 0:1].astype(jnp.bfloat16).astype(jnp.float32)
        for c in (1, 2):
            mc = m_ref[:, c:c + 1]
            take = mc < acc
            idx = jnp.where(take, j_ref[:, c:c + 1], idx)
            acc = jnp.where(take, mc.astype(jnp.bfloat16).astype(jnp.float32),
                            acc)
        out_ref[...] = idx


def kernel(h, codebook):
    B, T, D = h.shape
    K = codebook.shape[0]
    M = B * T
    h_flat = h.reshape(M, D)

    h_sq = jnp.sum(h_flat ** 2, axis=-1, keepdims=True)
    c_sq = jnp.sum(codebook ** 2, axis=-1, keepdims=True).T
    h_bf = h_flat.astype(jnp.bfloat16)
    cb_bf = codebook.astype(jnp.bfloat16)

    num_m = M // BM
    num_k = K // BK

    out = pl.pallas_call(
        _vq_block_kernel,
        grid=(num_m, num_k),
        in_specs=[
            pl.BlockSpec((BM, D), lambda m, k: (m, 0)),
            pl.BlockSpec((BK, D), lambda m, k: (k, 0)),
            pl.BlockSpec((BM, 1), lambda m, k: (m, 0)),
            pl.BlockSpec((1, BK), lambda m, k: (0, k)),
        ],
        out_specs=pl.BlockSpec((BM, 1), lambda m, k: (m, 0)),
        out_shape=jax.ShapeDtypeStruct((M, 1), jnp.int32),
        scratch_shapes=[
            pltpu.VMEM((BM, 3), jnp.float32),
            pltpu.VMEM((BM, 3), jnp.int32),
        ],
        compiler_params=pltpu.CompilerParams(
            dimension_semantics=("parallel", "arbitrary"),
        ),
    )(h_bf, cb_bf, h_sq, c_sq)
    return out.reshape(B, T)


# speedup vs baseline: 3.3175x; 1.3084x over previous
import jax
import jax.numpy as jnp
from jax.experimental import pallas as pl
from jax.experimental.pallas import tpu as pltpu

BM = 1024
BK = 1024
K_TOTAL = 8192
CHUNK = 2736
_NK = K_TOTAL // BK
_IMAX = 2 ** 31 - 1

_COVER = []
for _k in range(_NK):
    lo_code, hi_code = _k * BK, (_k + 1) * BK
    segs = []
    for _c in range(3):
        c_lo, c_hi = _c * CHUNK, min((_c + 1) * CHUNK, K_TOTAL)
        s_lo, s_hi = max(lo_code, c_lo), min(hi_code, c_hi)
        if s_lo < s_hi:
            segs.append((_c, s_lo - lo_code, s_hi - lo_code))
    _COVER.append(segs)


def _vq_block_kernel(h_ref, cb_ref, hsq_ref, csq_ref, out_ref,
                     v0_ref, v1_ref, v2_ref, j0_ref, j1_ref, j2_ref):
    k = pl.program_id(1)
    nk = pl.num_programs(1)
    vrefs = (v0_ref, v1_ref, v2_ref)
    jrefs = (j0_ref, j1_ref, j2_ref)

    scores = jax.lax.dot_general(
        h_ref[...], cb_ref[...], (((1,), (1,)), ((), ())),
        preferred_element_type=jnp.float32,
    )
    dist = (hsq_ref[...] + csq_ref[...]) - 2.0 * scores

    lane = jax.lax.broadcasted_iota(jnp.int32, (1, BK), 1)
    gidx = k * BK + lane

    @pl.when(k == 0)
    def _init():
        for vr, jr in zip(vrefs, jrefs):
            vr[...] = jnp.full(vr.shape, jnp.inf, jnp.float32)
            jr[...] = jnp.full(jr.shape, _IMAX, jnp.int32)

    for kk, segs in enumerate(_COVER):
        @pl.when(k == kk)
        def _upd(segs=segs):
            for c, lo, hi in segs:
                prev = vrefs[c][...]
                upd = dist < prev
                if lo > 0 or hi < BK:
                    upd = jnp.logical_and(upd, (lane >= lo) & (lane < hi))
                vrefs[c][...] = jnp.where(upd, dist, prev)
                jrefs[c][...] = jnp.where(upd, gidx, jrefs[c][...])

    @pl.when(k == nk - 1)
    def _emit():
        ms, js = [], []
        for c in range(3):
            vals = vrefs[c][...]
            m = jnp.min(vals, axis=1, keepdims=True)
            cand = jnp.where(vals == m, jrefs[c][...], _IMAX)
            j = jnp.min(cand, axis=1, keepdims=True)
            ms.append(m)
            js.append(j)
        idx = js[0]
        acc = ms[0].astype(jnp.bfloat16).astype(jnp.float32)
        for c in (1, 2):
            take = ms[c] < acc
            idx = jnp.where(take, js[c], idx)
            acc = jnp.where(take, ms[c].astype(jnp.bfloat16).astype(jnp.float32),
                            acc)
        out_ref[...] = idx


def kernel(h, codebook):
    B, T, D = h.shape
    K = codebook.shape[0]
    M = B * T
    h_flat = h.reshape(M, D)

    h_sq = jnp.sum(h_flat ** 2, axis=-1, keepdims=True)
    c_sq = jnp.sum(codebook ** 2, axis=-1, keepdims=True).T
    h_bf = h_flat.astype(jnp.bfloat16)
    cb_bf = codebook.astype(jnp.bfloat16)

    num_m = M // BM
    num_k = K // BK

    out = pl.pallas_call(
        _vq_block_kernel,
        grid=(num_m, num_k),
        in_specs=[
            pl.BlockSpec((BM, D), lambda m, k: (m, 0)),
            pl.BlockSpec((BK, D), lambda m, k: (k, 0)),
            pl.BlockSpec((BM, 1), lambda m, k: (m, 0)),
            pl.BlockSpec((1, BK), lambda m, k: (0, k)),
        ],
        out_specs=pl.BlockSpec((BM, 1), lambda m, k: (m, 0)),
        out_shape=jax.ShapeDtypeStruct((M, 1), jnp.int32),
        scratch_shapes=[
            pltpu.VMEM((BM, BK), jnp.float32),
            pltpu.VMEM((BM, BK), jnp.float32),
            pltpu.VMEM((BM, BK), jnp.float32),
            pltpu.VMEM((BM, BK), jnp.int32),
            pltpu.VMEM((BM, BK), jnp.int32),
            pltpu.VMEM((BM, BK), jnp.int32),
        ],
        compiler_params=pltpu.CompilerParams(
            dimension_semantics=("parallel", "arbitrary"),
        ),
    )(h_bf, cb_bf, h_sq, c_sq)
    return out.reshape(B, T)
